# Initial kernel scaffold; baseline (speedup 1.0000x reference)
import jax
import jax.numpy as jnp
from jax.experimental import pallas as pl


def kernel(a, W, H):
    raise NotImplementedError("write your pallas kernel here")


# speedup vs baseline: 319.5337x; 319.5337x over previous
import functools

import jax
import jax.numpy as jnp
from jax import lax
from jax.experimental import pallas as pl
from jax.experimental.pallas import tpu as pltpu
from jax.experimental.pallas import tpu_sc as plsc

_FAN_IN = 1024
_FAN_OUT = 1024
_K = 2048
_NW = 32
_ROW = _FAN_IN + 1
_PER_W = (_FAN_OUT // _NW) * _ROW
_ROWS_PER_W = _FAN_OUT // _NW
_NCB = _FAN_IN // 128 + 1
_G_TOTAL = _NCB * _FAN_OUT * 128


def _gather_body(w_hbm, h_hbm, g_hbm, w_v, h_v, g_v):
    wid = lax.axis_index("s") * 2 + lax.axis_index("c")
    pltpu.sync_copy(w_hbm, w_v)
    pltpu.sync_copy(h_hbm.at[pl.ds(wid * _PER_W, _PER_W)], h_v)

    @plsc.parallel_loop(0, _ROWS_PER_W, 1, unroll=2)
    def _(r):
        hbase = r * _ROW
        gbase = r * 128
        for c in range(8):
            for v in range(8):
                idx = h_v[pl.ds(hbase + c * 128 + v * 16, 16)]
                val = plsc.load_gather(w_v, [idx])
                g_v[pl.ds((c * _ROWS_PER_W) * 128 + gbase + v * 16, 16)] = val
        pos = jnp.full((16,), hbase + _FAN_IN, jnp.int32)
        bidx = plsc.load_gather(h_v, [pos])
        bval = plsc.load_gather(w_v, [bidx])
        for v in range(8):
            g_v[pl.ds((8 * _ROWS_PER_W) * 128 + gbase + v * 16, 16)] = bval

    for c in range(_NCB):
        blk = _ROWS_PER_W * 128
        pltpu.sync_copy(
            g_v.at[pl.ds(c * blk, blk)],
            g_hbm.at[pl.ds((c * _FAN_OUT + _ROWS_PER_W * wid) * 128, blk)],
        )


_gather = functools.partial(
    pl.kernel,
    mesh=plsc.VectorSubcoreMesh(core_axis_name="c", subcore_axis_name="s"),
    out_type=jax.ShapeDtypeStruct((_G_TOTAL,), jnp.float32),
    scratch_types=[
        pltpu.VMEM((_K,), jnp.float32),
        pltpu.VMEM((_PER_W,), jnp.int32),
        pltpu.VMEM((_NCB * _ROWS_PER_W * 128,), jnp.float32),
    ],
    compiler_params=pltpu.CompilerParams(needs_layout_passes=False),
)(_gather_body)


def _matmul_body(a_ref, g_ref, o_ref):
    a = a_ref[...]
    b = a.shape[0]
    acc = jnp.zeros((b, _FAN_OUT), jnp.float32)
    for c in range(8):
        acc += lax.dot_general(
            a[:, c * 128:(c + 1) * 128],
            g_ref[pl.ds(c * _FAN_OUT, _FAN_OUT), :],
            (((1,), (1,)), ((), ())),
            preferred_element_type=jnp.float32,
        )
    onehot = (lax.broadcasted_iota(jnp.int32, (b, 128), 1) == 0)
    acc += lax.dot_general(
        onehot.astype(jnp.float32),
        g_ref[pl.ds(8 * _FAN_OUT, _FAN_OUT), :],
        (((1,), (1,)), ((), ())),
        preferred_element_type=jnp.float32,
    )
    o_ref[...] = acc


def kernel(a, W, H):
    g = _gather(W, H.reshape(-1))
    g2 = g.reshape(_NCB * _FAN_OUT, 128)
    return pl.pallas_call(
        _matmul_body,
        out_shape=jax.ShapeDtypeStruct((a.shape[0], _FAN_OUT), jnp.float32),
    )(a, g2)
